# baseline (speedup 1.0000x reference)
import jax
import jax.numpy as jnp
from jax.experimental import pallas as pl
from jax.experimental.pallas import tpu as pltpu

T, B, D_IN, D_H = 512, 64, 512, 512
G4 = 4 * D_H
S = 16

_DN_T = (((1,), (1,)), ((), ()))


def _lstm_kernel(
    x_ref, wf_ref, wi_ref, wg_ref, wo_ref, b_ref,
    out_ref, cx_ref, hx_ref,
    h_ref, c_ref, wht_ref, pre_ref,
):
    k = pl.program_id(0)
    w_refs = (wf_ref, wi_ref, wg_ref, wo_ref)

    @pl.when(k == 0)
    def _():
        h_ref[...] = jnp.zeros_like(h_ref)
        c_ref[...] = jnp.zeros_like(c_ref)
        for q, w_ref in enumerate(w_refs):
            wht_ref[:, q * D_H : (q + 1) * D_H] = (
                w_ref[:, D_IN:][...].T.astype(jnp.bfloat16)
            )

    x2 = x_ref[...].reshape(S * B, D_IN)
    for q, w_ref in enumerate(w_refs):
        pre_ref[:, q * D_H : (q + 1) * D_H] = (
            jax.lax.dot_general(
                x2, w_ref[:, :D_IN][...], _DN_T,
                preferred_element_type=jnp.float32,
            )
            + b_ref[0, q * D_H : (q + 1) * D_H]
        ).astype(jnp.bfloat16)

    h = h_ref[...]
    c = c_ref[...]
    for s in range(S):
        hb = h.astype(jnp.bfloat16)
        row = slice(s * B, (s + 1) * B)

        def gate(q):
            return pre_ref[row, q * D_H : (q + 1) * D_H] + jnp.dot(
                hb, wht_ref[:, q * D_H : (q + 1) * D_H][...],
                preferred_element_type=jnp.float32,
            )

        f = 0.5 * jnp.tanh(0.5 * gate(0)) + 0.5
        i = 0.5 * jnp.tanh(0.5 * gate(1)) + 0.5
        g = jnp.tanh(gate(2))
        o = 0.5 * jnp.tanh(0.5 * gate(3)) + 0.5
        c = f * c + i * g
        h = o * jnp.tanh(c)
        out_ref[s] = h
    h_ref[...] = h
    c_ref[...] = c

    @pl.when(k == T // S - 1)
    def _():
        cx_ref[...] = c
        hx_ref[...] = h


def kernel(inputs, Wf, bf, Wi, bi, Wg, bg, Wo, bo):
    b = jnp.concatenate([bf, bi, bg, bo]).reshape(1, G4)

    w_spec = pl.BlockSpec((D_H, D_IN + D_H), lambda k: (0, 0))
    outputs, cx, hx = pl.pallas_call(
        _lstm_kernel,
        out_shape=(
            jax.ShapeDtypeStruct((T, B, D_H), jnp.float32),
            jax.ShapeDtypeStruct((B, D_H), jnp.float32),
            jax.ShapeDtypeStruct((B, D_H), jnp.float32),
        ),
        grid=(T // S,),
        in_specs=[
            pl.BlockSpec((S, B, D_IN), lambda k: (k, 0, 0)),
            w_spec, w_spec, w_spec, w_spec,
            pl.BlockSpec((1, G4), lambda k: (0, 0)),
        ],
        out_specs=(
            pl.BlockSpec((S, B, D_H), lambda k: (k, 0, 0)),
            pl.BlockSpec((B, D_H), lambda k: (0, 0)),
            pl.BlockSpec((B, D_H), lambda k: (0, 0)),
        ),
        scratch_shapes=[
            pltpu.VMEM((B, D_H), jnp.float32),
            pltpu.VMEM((B, D_H), jnp.float32),
            pltpu.VMEM((D_H, G4), jnp.bfloat16),
            pltpu.VMEM((S * B, G4), jnp.bfloat16),
        ],
        compiler_params=pltpu.CompilerParams(
            dimension_semantics=("arbitrary",),
        ),
        name="lstm_fused",
    )(inputs, Wf, Wi, Wg, Wo, b)

    return outputs, (hx, cx)


# speedup vs baseline: 1.0073x; 1.0073x over previous
import jax
import jax.numpy as jnp
from jax.experimental import pallas as pl
from jax.experimental.pallas import tpu as pltpu

T, B, D_IN, D_H = 512, 64, 512, 512
G4 = 4 * D_H
S = 16

_DN_T = (((1,), (1,)), ((), ()))


def _lstm_kernel(
    x_ref, wf_ref, wi_ref, wg_ref, wo_ref, b_ref,
    out_ref, cx_ref, hx_ref,
    h_ref, c_ref, wht_ref, pre_ref,
):
    k = pl.program_id(0)
    w_refs = (wf_ref, wi_ref, wg_ref, wo_ref)

    @pl.when(k == 0)
    def _():
        h_ref[...] = jnp.zeros_like(h_ref)
        c_ref[...] = jnp.zeros_like(c_ref)
        for q, w_ref in enumerate(w_refs):
            wht_ref[:, q * D_H : (q + 1) * D_H] = (
                w_ref[:, D_IN:][...].T.astype(jnp.bfloat16)
            )

    x2 = x_ref[...].reshape(S * B, D_IN)
    for q, w_ref in enumerate(w_refs):
        pre_ref[:, q * D_H : (q + 1) * D_H] = (
            jax.lax.dot_general(
                x2, w_ref[:, :D_IN][...], _DN_T,
                preferred_element_type=jnp.float32,
            )
            + b_ref[0, q * D_H : (q + 1) * D_H]
        )

    h = h_ref[...]
    c = c_ref[...]
    for s in range(S):
        hb = h.astype(jnp.bfloat16)
        row = slice(s * B, (s + 1) * B)

        def gate(q):
            return pre_ref[row, q * D_H : (q + 1) * D_H] + jnp.dot(
                hb, wht_ref[:, q * D_H : (q + 1) * D_H][...],
                preferred_element_type=jnp.float32,
            )

        f = 0.5 * jnp.tanh(0.5 * gate(0)) + 0.5
        i = 0.5 * jnp.tanh(0.5 * gate(1)) + 0.5
        g = jnp.tanh(gate(2))
        o = 0.5 * jnp.tanh(0.5 * gate(3)) + 0.5
        c = f * c + i * g
        h = o * jnp.tanh(c)
        out_ref[s] = h
    h_ref[...] = h
    c_ref[...] = c

    @pl.when(k == T // S - 1)
    def _():
        cx_ref[...] = c
        hx_ref[...] = h


def kernel(inputs, Wf, bf, Wi, bi, Wg, bg, Wo, bo):
    b = jnp.concatenate([bf, bi, bg, bo]).reshape(1, G4)

    w_spec = pl.BlockSpec((D_H, D_IN + D_H), lambda k: (0, 0))
    outputs, cx, hx = pl.pallas_call(
        _lstm_kernel,
        out_shape=(
            jax.ShapeDtypeStruct((T, B, D_H), jnp.float32),
            jax.ShapeDtypeStruct((B, D_H), jnp.float32),
            jax.ShapeDtypeStruct((B, D_H), jnp.float32),
        ),
        grid=(T // S,),
        in_specs=[
            pl.BlockSpec((S, B, D_IN), lambda k: (k, 0, 0)),
            w_spec, w_spec, w_spec, w_spec,
            pl.BlockSpec((1, G4), lambda k: (0, 0)),
        ],
        out_specs=(
            pl.BlockSpec((S, B, D_H), lambda k: (k, 0, 0)),
            pl.BlockSpec((B, D_H), lambda k: (0, 0)),
            pl.BlockSpec((B, D_H), lambda k: (0, 0)),
        ),
        scratch_shapes=[
            pltpu.VMEM((B, D_H), jnp.float32),
            pltpu.VMEM((B, D_H), jnp.float32),
            pltpu.VMEM((D_H, G4), jnp.bfloat16),
            pltpu.VMEM((S * B, G4), jnp.float32),
        ],
        compiler_params=pltpu.CompilerParams(
            dimension_semantics=("arbitrary",),
        ),
        name="lstm_fused",
    )(inputs, Wf, Wi, Wg, Wo, b)

    return outputs, (hx, cx)
